# baseline (speedup 1.0000x reference)
import functools

import jax
import jax.numpy as jnp
from jax import lax
from jax.experimental import pallas as pl
from jax.experimental.pallas import tpu as pltpu
from jax.experimental.pallas import tpu_sc as plsc

N = 10000
E = 320000
D = 128
G = 64
O = 64

NC, NS = 2, 16
NW = NC * NS
NP = 10240
RPT = NP // NS
C = 128
NCH = 80
EPT = NCH * C
EPAD = EPT * NW


@functools.cache
def _sc_kernels():
    mesh = plsc.VectorSubcoreMesh(core_axis_name="c", subcore_axis_name="s",
                                  num_cores=NC, num_subcores=NS)

    @functools.partial(
        pl.kernel,
        out_type=jax.ShapeDtypeStruct((NC, NP, D), jnp.float32),
        mesh=mesh,
        scratch_types=[
            pltpu.VMEM((2, C), jnp.int32),
            pltpu.VMEM((2, C), jnp.int32),
            pltpu.VMEM((C, D), jnp.float32),
            pltpu.SemaphoreType.DMA,
            pltpu.SemaphoreType.DMA,
            pltpu.VMEM_SHARED((NP, D), jnp.float32),
        ],
    )
    def deg_kernel(eidx_hbm, ones_hbm, zeros_hbm, out_hbm, ib0, ib1, onesv,
                   isem0, isem1, acc):
        cid = lax.axis_index("c")
        sid = lax.axis_index("s")
        pltpu.sync_copy(ones_hbm, onesv)
        pltpu.sync_copy(zeros_hbm.at[pl.ds(sid * RPT, RPT)],
                        acc.at[pl.ds(sid * RPT, RPT)])
        pltpu.async_copy(eidx_hbm.at[cid, sid, 0], ib0, isem0)
        pltpu.async_copy(eidx_hbm.at[cid, sid, 1], ib1, isem1)
        plsc.subcore_barrier()

        def half(j, ibA, isemA):
            pltpu.make_async_copy(eidx_hbm.at[cid, sid, j], ibA, isemA).wait()
            pltpu.sync_copy(onesv, acc.at[ibA.at[1]], add=True)

            @pl.when(j + 2 < NCH)
            def _():
                pltpu.async_copy(eidx_hbm.at[cid, sid, j + 2], ibA, isemA)

        def body(i, carry):
            half(2 * i, ib0, isem0)
            half(2 * i + 1, ib1, isem1)
            return carry

        lax.fori_loop(0, NCH // 2, body, 0)
        plsc.subcore_barrier()
        pltpu.sync_copy(acc.at[pl.ds(sid * RPT, RPT)],
                        out_hbm.at[cid, pl.ds(sid * RPT, RPT)])

    @functools.partial(
        pl.kernel,
        out_type=jax.ShapeDtypeStruct((NC, NP, D), jnp.float32),
        mesh=mesh,
        scratch_types=[
            pltpu.VMEM((2, C), jnp.int32),
            pltpu.VMEM((2, C), jnp.int32),
            pltpu.VMEM((C, D), jnp.float32),
            pltpu.VMEM((C, D), jnp.float32),
            pltpu.SemaphoreType.DMA,
            pltpu.SemaphoreType.DMA,
            pltpu.SemaphoreType.DMA,
            pltpu.SemaphoreType.DMA,
            pltpu.VMEM_SHARED((NP, D), jnp.float32),
        ],
    )
    def edge_kernel(g_hbm, eidx_hbm, zeros_hbm, out_hbm,
                    ib0, ib1, rows0, rows1, isem0, isem1, gsem0, gsem1, acc):
        cid = lax.axis_index("c")
        sid = lax.axis_index("s")
        pltpu.sync_copy(zeros_hbm.at[pl.ds(sid * RPT, RPT)],
                        acc.at[pl.ds(sid * RPT, RPT)])
        pltpu.sync_copy(eidx_hbm.at[cid, sid, 0], ib0)
        pltpu.async_copy(eidx_hbm.at[cid, sid, 1], ib1, isem1)
        pltpu.async_copy(g_hbm.at[ib0.at[0]], rows0, gsem0)
        plsc.subcore_barrier()

        def half(j, ibA, ibB, rowsA, rowsB, isemA, isemB, gsemA, gsemB):
            @pl.when(j + 1 < NCH)
            def _():
                pltpu.make_async_copy(eidx_hbm.at[cid, sid, j + 1], ibB,
                                      isemB).wait()
                pltpu.async_copy(g_hbm.at[ibB.at[0]], rowsB, gsemB)

            pltpu.make_async_copy(g_hbm.at[ibA.at[0]], rowsA, gsemA).wait()
            pltpu.sync_copy(rowsA, acc.at[ibA.at[1]], add=True)

            @pl.when(j + 2 < NCH)
            def _():
                pltpu.async_copy(eidx_hbm.at[cid, sid, j + 2], ibA, isemA)

        def body(i, carry):
            j0 = 2 * i
            half(j0, ib0, ib1, rows0, rows1, isem0, isem1, gsem0, gsem1)
            half(j0 + 1, ib1, ib0, rows1, rows0, isem1, isem0, gsem1, gsem0)
            return carry

        lax.fori_loop(0, NCH // 2, body, 0)
        plsc.subcore_barrier()
        pltpu.sync_copy(acc.at[pl.ds(sid * RPT, RPT)],
                        out_hbm.at[cid, pl.ds(sid * RPT, RPT)])

    return deg_kernel, edge_kernel


BN = 1000
GRID = N // BN


def _dinv_of(deg_ref):
    d = deg_ref[0, :, 0] + deg_ref[1, :, 0] + 1.0
    return lax.rsqrt(d)[:, None]


def _mm1_body(deg_ref, x_ref, w_ref, o_ref):
    h = jnp.dot(x_ref[...], w_ref[...], preferred_element_type=jnp.float32)
    o_ref[...] = _dinv_of(deg_ref) * h


def _mmn_body(deg_ref, acc_ref, g_ref, w_ref, b_ref, o_ref):
    dinv = _dinv_of(deg_ref)
    xin = dinv * (acc_ref[0] + acc_ref[1] + g_ref[...]) + b_ref[...]
    xin = jnp.maximum(xin, 0.0)
    o_ref[...] = dinv * jnp.dot(xin, w_ref[...],
                                preferred_element_type=jnp.float32)


def _pool_body(deg_ref, acc_ref, g_ref, b_ref, batch_ref,
               wo1_ref, bo1_ref, wo2_ref, bo2_ref, o_ref,
               sums_ref, cnt_ref):
    step = pl.program_id(0)

    @pl.when(step == 0)
    def _():
        sums_ref[...] = jnp.zeros_like(sums_ref)
        cnt_ref[...] = jnp.zeros_like(cnt_ref)

    dinv = _dinv_of(deg_ref)
    h3 = dinv * (acc_ref[0] + acc_ref[1] + g_ref[...]) + b_ref[...]
    gids = lax.broadcasted_iota(jnp.int32, (1, G), 1).astype(jnp.float32)
    mask = (batch_ref[...] == gids).astype(jnp.float32)
    sums_ref[...] += lax.dot_general(
        mask, h3, (((0,), (0,)), ((), ())),
        preferred_element_type=jnp.float32)
    cnt_ref[...] += jnp.sum(mask, axis=0)[None, :]

    @pl.when(step == GRID - 1)
    def _():
        pooled = sums_ref[...] / jnp.maximum(cnt_ref[...], 1.0)[0][:, None]
        z = jnp.maximum(
            jnp.dot(pooled, wo1_ref[...],
                    preferred_element_type=jnp.float32) + bo1_ref[...], 0.0)
        o_ref[...] = jnp.dot(z, wo2_ref[...],
                             preferred_element_type=jnp.float32) + bo2_ref[...]


_deg_spec = pl.BlockSpec((NC, BN, D), lambda i: (0, i, 0))
_acc_spec = pl.BlockSpec((NC, BN, D), lambda i: (0, i, 0))
_row_spec = pl.BlockSpec((BN, D), lambda i: (i, 0))


def _full(shape):
    return pl.BlockSpec(shape, lambda i: tuple(0 for _ in shape))


_mm1 = pl.pallas_call(
    _mm1_body,
    grid=(GRID,),
    in_specs=[_deg_spec, _row_spec, _full((D, D))],
    out_specs=_row_spec,
    out_shape=jax.ShapeDtypeStruct((N, D), jnp.float32),
)

_mmn = pl.pallas_call(
    _mmn_body,
    grid=(GRID,),
    in_specs=[_deg_spec, _acc_spec, _row_spec, _full((D, D)), _full((1, D))],
    out_specs=_row_spec,
    out_shape=jax.ShapeDtypeStruct((N, D), jnp.float32),
)

_pool = pl.pallas_call(
    _pool_body,
    grid=(GRID,),
    in_specs=[_deg_spec, _acc_spec, _row_spec, _full((1, D)),
              pl.BlockSpec((BN, 1), lambda i: (i, 0)),
              _full((D, G)), _full((1, G)), _full((G, O)), _full((1, O))],
    out_specs=_full((G, O)),
    out_shape=jax.ShapeDtypeStruct((G, O), jnp.float32),
    scratch_shapes=[pltpu.VMEM((G, D), jnp.float32),
                    pltpu.VMEM((1, G), jnp.float32)],
)


def kernel(x, edge_index, batch, W1, b1, W2, b2, W3, b3, Wo1, bo1, Wo2, bo2):
    src = edge_index[0].astype(jnp.int32)
    dst = edge_index[1].astype(jnp.int32)
    pad = EPAD - E
    src = jnp.concatenate([src, jnp.zeros((pad,), jnp.int32)])
    dst = jnp.concatenate([dst, jnp.full((pad,), NP - 1, jnp.int32)])
    eidx = jnp.stack([src.reshape(NC, NS, NCH, C),
                      dst.reshape(NC, NS, NCH, C)], axis=3)

    zerosD = jnp.zeros((NP, D), jnp.float32)
    batchf = batch.astype(jnp.float32).reshape(N, 1)
    b1r = b1.reshape(1, D)
    b2r = b2.reshape(1, D)
    b3r = b3.reshape(1, D)
    bo1r = bo1.reshape(1, G)
    bo2r = bo2.reshape(1, O)

    deg_kernel, edge_kernel = _sc_kernels()
    deg = deg_kernel(eidx, jnp.ones((C, D), jnp.float32), zerosD)
    g1 = _mm1(deg, x, W1)
    acc1 = edge_kernel(g1, eidx, zerosD)
    g2 = _mmn(deg, acc1, g1, W2, b1r)
    acc2 = edge_kernel(g2, eidx, zerosD)
    g3 = _mmn(deg, acc2, g2, W3, b2r)
    acc3 = edge_kernel(g3, eidx, zerosD)
    return _pool(deg, acc3, g3, b3r, batchf, Wo1, bo1r, Wo2, bo2r)


# speedup vs baseline: 2.0160x; 2.0160x over previous
import functools

import jax
import jax.numpy as jnp
from jax import lax
from jax.experimental import pallas as pl
from jax.experimental.pallas import tpu as pltpu
from jax.experimental.pallas import tpu_sc as plsc

N = 10000
E = 320000
D = 128
G = 64
O = 64

NC, NS = 2, 16
NW = NC * NS
NP = 10240
RPT = NP // NS
C = 112
NCH = 90
EPT = NCH * C
EPAD = EPT * NW


@functools.cache
def _sc_kernels():
    mesh = plsc.VectorSubcoreMesh(core_axis_name="c", subcore_axis_name="s",
                                  num_cores=NC, num_subcores=NS)

    @functools.partial(
        pl.kernel,
        out_type=jax.ShapeDtypeStruct((NC, NP, D), jnp.float32),
        mesh=mesh,
        scratch_types=[
            pltpu.VMEM((2, C), jnp.int32),
            pltpu.VMEM((2, C), jnp.int32),
            pltpu.VMEM((C, D), jnp.float32),
            pltpu.SemaphoreType.DMA,
            pltpu.SemaphoreType.DMA,
            pltpu.VMEM_SHARED((NP, D), jnp.float32),
        ],
    )
    def deg_kernel(eidx_hbm, ones_hbm, zeros_hbm, out_hbm, ib0, ib1, onesv,
                   isem0, isem1, acc):
        cid = lax.axis_index("c")
        sid = lax.axis_index("s")
        pltpu.sync_copy(ones_hbm, onesv)
        pltpu.sync_copy(zeros_hbm.at[pl.ds(sid * RPT, RPT)],
                        acc.at[pl.ds(sid * RPT, RPT)])
        pltpu.async_copy(eidx_hbm.at[cid, sid, 0], ib0, isem0)
        pltpu.async_copy(eidx_hbm.at[cid, sid, 1], ib1, isem1)
        plsc.subcore_barrier()

        def half(j, ibA, isemA):
            pltpu.make_async_copy(eidx_hbm.at[cid, sid, j], ibA, isemA).wait()
            pltpu.sync_copy(onesv, acc.at[ibA.at[1]], add=True)

            @pl.when(j + 2 < NCH)
            def _():
                pltpu.async_copy(eidx_hbm.at[cid, sid, j + 2], ibA, isemA)

        def body(i, carry):
            half(2 * i, ib0, isem0)
            half(2 * i + 1, ib1, isem1)
            return carry

        lax.fori_loop(0, NCH // 2, body, 0)
        plsc.subcore_barrier()
        pltpu.sync_copy(acc.at[pl.ds(sid * RPT, RPT)],
                        out_hbm.at[cid, pl.ds(sid * RPT, RPT)])

    @functools.partial(
        pl.kernel,
        out_type=jax.ShapeDtypeStruct((NC, NP, D), jnp.float32),
        mesh=mesh,
        scratch_types=[
            pltpu.VMEM((2, C), jnp.int32),
            pltpu.VMEM((2, C), jnp.int32),
            pltpu.VMEM((C, D), jnp.float32),
            pltpu.VMEM((C, D), jnp.float32),
            pltpu.SemaphoreType.DMA,
            pltpu.SemaphoreType.DMA,
            pltpu.SemaphoreType.DMA,
            pltpu.SemaphoreType.DMA,
            pltpu.VMEM_SHARED((NP, D), jnp.float32),
        ],
    )
    def edge_kernel(g_hbm, eidx_hbm, zeros_hbm, out_hbm,
                    ib0, ib1, rows0, rows1, isem0, isem1, gsem0, gsem1, acc):
        cid = lax.axis_index("c")
        sid = lax.axis_index("s")
        pltpu.sync_copy(zeros_hbm.at[pl.ds(sid * RPT, RPT)],
                        acc.at[pl.ds(sid * RPT, RPT)])
        pltpu.sync_copy(eidx_hbm.at[cid, sid, 0], ib0)
        pltpu.async_copy(eidx_hbm.at[cid, sid, 1], ib1, isem1)
        pltpu.async_copy(g_hbm.at[ib0.at[0]], rows0, gsem0)
        plsc.subcore_barrier()

        def half(j, ibA, ibB, rowsA, rowsB, isemA, isemB, gsemA, gsemB):
            @pl.when(j + 1 < NCH)
            def _():
                pltpu.make_async_copy(eidx_hbm.at[cid, sid, j + 1], ibB,
                                      isemB).wait()
                pltpu.async_copy(g_hbm.at[ibB.at[0]], rowsB, gsemB)

            pltpu.make_async_copy(g_hbm.at[ibA.at[0]], rowsA, gsemA).wait()
            pltpu.sync_copy(rowsA, acc.at[ibA.at[1]], add=True)

            @pl.when(j + 2 < NCH)
            def _():
                pltpu.async_copy(eidx_hbm.at[cid, sid, j + 2], ibA, isemA)

        def body(i, carry):
            j0 = 2 * i
            half(j0, ib0, ib1, rows0, rows1, isem0, isem1, gsem0, gsem1)
            half(j0 + 1, ib1, ib0, rows1, rows0, isem1, isem0, gsem1, gsem0)
            return carry

        lax.fori_loop(0, NCH // 2, body, 0)
        plsc.subcore_barrier()
        pltpu.sync_copy(acc.at[pl.ds(sid * RPT, RPT)],
                        out_hbm.at[cid, pl.ds(sid * RPT, RPT)])

    return deg_kernel, edge_kernel


BN = 1000
GRID = N // BN


def _dinv_of(deg_ref):
    d = deg_ref[0, :, 0] + deg_ref[1, :, 0] + 1.0
    return lax.rsqrt(d)[:, None]


def _mm1_body(deg_ref, x_ref, w_ref, o_ref):
    h = jnp.dot(x_ref[...], w_ref[...], preferred_element_type=jnp.float32)
    o_ref[...] = _dinv_of(deg_ref) * h


def _mmn_body(deg_ref, acc_ref, g_ref, w_ref, b_ref, o_ref):
    dinv = _dinv_of(deg_ref)
    xin = dinv * (acc_ref[0] + acc_ref[1] + g_ref[...]) + b_ref[...]
    xin = jnp.maximum(xin, 0.0)
    o_ref[...] = dinv * jnp.dot(xin, w_ref[...],
                                preferred_element_type=jnp.float32)


def _pool_body(deg_ref, acc_ref, g_ref, b_ref, batch_ref,
               wo1_ref, bo1_ref, wo2_ref, bo2_ref, o_ref,
               sums_ref, cnt_ref):
    step = pl.program_id(0)

    @pl.when(step == 0)
    def _():
        sums_ref[...] = jnp.zeros_like(sums_ref)
        cnt_ref[...] = jnp.zeros_like(cnt_ref)

    dinv = _dinv_of(deg_ref)
    h3 = dinv * (acc_ref[0] + acc_ref[1] + g_ref[...]) + b_ref[...]
    gids = lax.broadcasted_iota(jnp.int32, (1, G), 1).astype(jnp.float32)
    mask = (batch_ref[...] == gids).astype(jnp.float32)
    sums_ref[...] += lax.dot_general(
        mask, h3, (((0,), (0,)), ((), ())),
        preferred_element_type=jnp.float32)
    cnt_ref[...] += jnp.sum(mask, axis=0)[None, :]

    @pl.when(step == GRID - 1)
    def _():
        pooled = sums_ref[...] / jnp.maximum(cnt_ref[...], 1.0)[0][:, None]
        z = jnp.maximum(
            jnp.dot(pooled, wo1_ref[...],
                    preferred_element_type=jnp.float32) + bo1_ref[...], 0.0)
        o_ref[...] = jnp.dot(z, wo2_ref[...],
                             preferred_element_type=jnp.float32) + bo2_ref[...]


_deg_spec = pl.BlockSpec((NC, BN, D), lambda i: (0, i, 0))
_acc_spec = pl.BlockSpec((NC, BN, D), lambda i: (0, i, 0))
_row_spec = pl.BlockSpec((BN, D), lambda i: (i, 0))


def _full(shape):
    return pl.BlockSpec(shape, lambda i: tuple(0 for _ in shape))


_mm1 = pl.pallas_call(
    _mm1_body,
    grid=(GRID,),
    in_specs=[_deg_spec, _row_spec, _full((D, D))],
    out_specs=_row_spec,
    out_shape=jax.ShapeDtypeStruct((N, D), jnp.float32),
)

_mmn = pl.pallas_call(
    _mmn_body,
    grid=(GRID,),
    in_specs=[_deg_spec, _acc_spec, _row_spec, _full((D, D)), _full((1, D))],
    out_specs=_row_spec,
    out_shape=jax.ShapeDtypeStruct((N, D), jnp.float32),
)

_pool = pl.pallas_call(
    _pool_body,
    grid=(GRID,),
    in_specs=[_deg_spec, _acc_spec, _row_spec, _full((1, D)),
              pl.BlockSpec((BN, 1), lambda i: (i, 0)),
              _full((D, G)), _full((1, G)), _full((G, O)), _full((1, O))],
    out_specs=_full((G, O)),
    out_shape=jax.ShapeDtypeStruct((G, O), jnp.float32),
    scratch_shapes=[pltpu.VMEM((G, D), jnp.float32),
                    pltpu.VMEM((1, G), jnp.float32)],
)


def kernel(x, edge_index, batch, W1, b1, W2, b2, W3, b3, Wo1, bo1, Wo2, bo2):
    src = edge_index[0].astype(jnp.int32)
    dst = edge_index[1].astype(jnp.int32)
    pad = EPAD - E
    src = jnp.concatenate([src, jnp.zeros((pad,), jnp.int32)])
    dst = jnp.concatenate([dst, jnp.full((pad,), NP - 1, jnp.int32)])
    eidx = jnp.stack([src.reshape(NC, NS, NCH, C),
                      dst.reshape(NC, NS, NCH, C)], axis=3)

    zerosD = jnp.zeros((NP, D), jnp.float32)
    batchf = batch.astype(jnp.float32).reshape(N, 1)
    b1r = b1.reshape(1, D)
    b2r = b2.reshape(1, D)
    b3r = b3.reshape(1, D)
    bo1r = bo1.reshape(1, G)
    bo2r = bo2.reshape(1, O)

    deg_kernel, edge_kernel = _sc_kernels()
    deg = deg_kernel(eidx, jnp.ones((C, D), jnp.float32), zerosD)
    g1 = _mm1(deg, x, W1)
    acc1 = edge_kernel(g1, eidx, zerosD)
    g2 = _mmn(deg, acc1, g1, W2, b1r)
    acc2 = edge_kernel(g2, eidx, zerosD)
    g3 = _mmn(deg, acc2, g2, W3, b2r)
    acc3 = edge_kernel(g3, eidx, zerosD)
    return _pool(deg, acc3, g3, b3r, batchf, Wo1, bo1r, Wo2, bo2r)


# speedup vs baseline: 2.0795x; 1.0315x over previous
import functools

import jax
import jax.numpy as jnp
from jax import lax
from jax.experimental import pallas as pl
from jax.experimental.pallas import tpu as pltpu
from jax.experimental.pallas import tpu_sc as plsc

N = 10000
E = 320000
D = 128
G = 64
O = 64

NC, NS = 2, 16
NW = NC * NS
NP = 10240
RPT = NP // NS
C = 120
NCH = 84
EPT = NCH * C
EPAD = EPT * NW


@functools.cache
def _sc_kernels():
    mesh = plsc.VectorSubcoreMesh(core_axis_name="c", subcore_axis_name="s",
                                  num_cores=NC, num_subcores=NS)

    @functools.partial(
        pl.kernel,
        out_type=jax.ShapeDtypeStruct((NC, NP, D), jnp.float32),
        mesh=mesh,
        scratch_types=[
            pltpu.VMEM((2, C), jnp.int32),
            pltpu.VMEM((2, C), jnp.int32),
            pltpu.VMEM((C, D), jnp.float32),
            pltpu.SemaphoreType.DMA,
            pltpu.SemaphoreType.DMA,
            pltpu.VMEM_SHARED((NP, D), jnp.float32),
        ],
    )
    def deg_kernel(eidx_hbm, ones_hbm, zeros_hbm, out_hbm, ib0, ib1, onesv,
                   isem0, isem1, acc):
        cid = lax.axis_index("c")
        sid = lax.axis_index("s")
        pltpu.sync_copy(ones_hbm, onesv)
        pltpu.sync_copy(zeros_hbm.at[pl.ds(sid * RPT, RPT)],
                        acc.at[pl.ds(sid * RPT, RPT)])
        pltpu.async_copy(eidx_hbm.at[cid, sid, 0], ib0, isem0)
        pltpu.async_copy(eidx_hbm.at[cid, sid, 1], ib1, isem1)
        plsc.subcore_barrier()

        def half(j, ibA, isemA):
            pltpu.make_async_copy(eidx_hbm.at[cid, sid, j], ibA, isemA).wait()
            pltpu.sync_copy(onesv, acc.at[ibA.at[1]], add=True)

            @pl.when(j + 2 < NCH)
            def _():
                pltpu.async_copy(eidx_hbm.at[cid, sid, j + 2], ibA, isemA)

        def body(i, carry):
            half(2 * i, ib0, isem0)
            half(2 * i + 1, ib1, isem1)
            return carry

        lax.fori_loop(0, NCH // 2, body, 0)
        plsc.subcore_barrier()
        pltpu.sync_copy(acc.at[pl.ds(sid * RPT, RPT)],
                        out_hbm.at[cid, pl.ds(sid * RPT, RPT)])

    @functools.partial(
        pl.kernel,
        out_type=jax.ShapeDtypeStruct((NC, NP, D), jnp.float32),
        mesh=mesh,
        scratch_types=[
            pltpu.VMEM((2, C), jnp.int32),
            pltpu.VMEM((2, C), jnp.int32),
            pltpu.VMEM((C, D), jnp.float32),
            pltpu.VMEM((C, D), jnp.float32),
            pltpu.SemaphoreType.DMA,
            pltpu.SemaphoreType.DMA,
            pltpu.SemaphoreType.DMA,
            pltpu.SemaphoreType.DMA,
            pltpu.VMEM_SHARED((NP, D), jnp.float32),
        ],
    )
    def edge_kernel(g_hbm, eidx_hbm, zeros_hbm, out_hbm,
                    ib0, ib1, rows0, rows1, isem0, isem1, gsem0, gsem1, acc):
        cid = lax.axis_index("c")
        sid = lax.axis_index("s")
        pltpu.sync_copy(zeros_hbm.at[pl.ds(sid * RPT, RPT)],
                        acc.at[pl.ds(sid * RPT, RPT)])
        pltpu.sync_copy(eidx_hbm.at[cid, sid, 0], ib0)
        pltpu.async_copy(eidx_hbm.at[cid, sid, 1], ib1, isem1)
        pltpu.async_copy(g_hbm.at[ib0.at[0]], rows0, gsem0)
        plsc.subcore_barrier()

        def half(j, ibA, ibB, rowsA, rowsB, isemA, isemB, gsemA, gsemB):
            @pl.when(j + 1 < NCH)
            def _():
                pltpu.make_async_copy(eidx_hbm.at[cid, sid, j + 1], ibB,
                                      isemB).wait()
                pltpu.async_copy(g_hbm.at[ibB.at[0]], rowsB, gsemB)

            pltpu.make_async_copy(g_hbm.at[ibA.at[0]], rowsA, gsemA).wait()
            pltpu.sync_copy(rowsA, acc.at[ibA.at[1]], add=True)

            @pl.when(j + 2 < NCH)
            def _():
                pltpu.async_copy(eidx_hbm.at[cid, sid, j + 2], ibA, isemA)

        def body(i, carry):
            j0 = 2 * i
            half(j0, ib0, ib1, rows0, rows1, isem0, isem1, gsem0, gsem1)
            half(j0 + 1, ib1, ib0, rows1, rows0, isem1, isem0, gsem1, gsem0)
            return carry

        lax.fori_loop(0, NCH // 2, body, 0)
        plsc.subcore_barrier()
        pltpu.sync_copy(acc.at[pl.ds(sid * RPT, RPT)],
                        out_hbm.at[cid, pl.ds(sid * RPT, RPT)])

    return deg_kernel, edge_kernel


BN = 1000
GRID = N // BN


def _dinv_of(deg_ref):
    d = deg_ref[0, :, 0] + deg_ref[1, :, 0] + 1.0
    return lax.rsqrt(d)[:, None]


def _mm1_body(deg_ref, x_ref, w_ref, o_ref):
    h = jnp.dot(x_ref[...], w_ref[...], preferred_element_type=jnp.float32)
    o_ref[...] = _dinv_of(deg_ref) * h


def _mmn_body(deg_ref, acc_ref, g_ref, w_ref, b_ref, o_ref):
    dinv = _dinv_of(deg_ref)
    xin = dinv * (acc_ref[0] + acc_ref[1] + g_ref[...]) + b_ref[...]
    xin = jnp.maximum(xin, 0.0)
    o_ref[...] = dinv * jnp.dot(xin, w_ref[...],
                                preferred_element_type=jnp.float32)


def _pool_body(deg_ref, acc_ref, g_ref, b_ref, batch_ref,
               wo1_ref, bo1_ref, wo2_ref, bo2_ref, o_ref,
               sums_ref, cnt_ref):
    step = pl.program_id(0)

    @pl.when(step == 0)
    def _():
        sums_ref[...] = jnp.zeros_like(sums_ref)
        cnt_ref[...] = jnp.zeros_like(cnt_ref)

    dinv = _dinv_of(deg_ref)
    h3 = dinv * (acc_ref[0] + acc_ref[1] + g_ref[...]) + b_ref[...]
    gids = lax.broadcasted_iota(jnp.int32, (1, G), 1).astype(jnp.float32)
    mask = (batch_ref[...] == gids).astype(jnp.float32)
    sums_ref[...] += lax.dot_general(
        mask, h3, (((0,), (0,)), ((), ())),
        preferred_element_type=jnp.float32)
    cnt_ref[...] += jnp.sum(mask, axis=0)[None, :]

    @pl.when(step == GRID - 1)
    def _():
        pooled = sums_ref[...] / jnp.maximum(cnt_ref[...], 1.0)[0][:, None]
        z = jnp.maximum(
            jnp.dot(pooled, wo1_ref[...],
                    preferred_element_type=jnp.float32) + bo1_ref[...], 0.0)
        o_ref[...] = jnp.dot(z, wo2_ref[...],
                             preferred_element_type=jnp.float32) + bo2_ref[...]


_deg_spec = pl.BlockSpec((NC, BN, D), lambda i: (0, i, 0))
_acc_spec = pl.BlockSpec((NC, BN, D), lambda i: (0, i, 0))
_row_spec = pl.BlockSpec((BN, D), lambda i: (i, 0))


def _full(shape):
    return pl.BlockSpec(shape, lambda i: tuple(0 for _ in shape))


_mm1 = pl.pallas_call(
    _mm1_body,
    grid=(GRID,),
    in_specs=[_deg_spec, _row_spec, _full((D, D))],
    out_specs=_row_spec,
    out_shape=jax.ShapeDtypeStruct((N, D), jnp.float32),
)

_mmn = pl.pallas_call(
    _mmn_body,
    grid=(GRID,),
    in_specs=[_deg_spec, _acc_spec, _row_spec, _full((D, D)), _full((1, D))],
    out_specs=_row_spec,
    out_shape=jax.ShapeDtypeStruct((N, D), jnp.float32),
)

_pool = pl.pallas_call(
    _pool_body,
    grid=(GRID,),
    in_specs=[_deg_spec, _acc_spec, _row_spec, _full((1, D)),
              pl.BlockSpec((BN, 1), lambda i: (i, 0)),
              _full((D, G)), _full((1, G)), _full((G, O)), _full((1, O))],
    out_specs=_full((G, O)),
    out_shape=jax.ShapeDtypeStruct((G, O), jnp.float32),
    scratch_shapes=[pltpu.VMEM((G, D), jnp.float32),
                    pltpu.VMEM((1, G), jnp.float32)],
)


def kernel(x, edge_index, batch, W1, b1, W2, b2, W3, b3, Wo1, bo1, Wo2, bo2):
    src = edge_index[0].astype(jnp.int32)
    dst = edge_index[1].astype(jnp.int32)
    pad = EPAD - E
    src = jnp.concatenate([src, jnp.zeros((pad,), jnp.int32)])
    dst = jnp.concatenate([dst, jnp.full((pad,), NP - 1, jnp.int32)])
    eidx = jnp.stack([src.reshape(NC, NS, NCH, C),
                      dst.reshape(NC, NS, NCH, C)], axis=3)

    zerosD = jnp.zeros((NP, D), jnp.float32)
    batchf = batch.astype(jnp.float32).reshape(N, 1)
    b1r = b1.reshape(1, D)
    b2r = b2.reshape(1, D)
    b3r = b3.reshape(1, D)
    bo1r = bo1.reshape(1, G)
    bo2r = bo2.reshape(1, O)

    deg_kernel, edge_kernel = _sc_kernels()
    deg = deg_kernel(eidx, jnp.ones((C, D), jnp.float32), zerosD)
    g1 = _mm1(deg, x, W1)
    acc1 = edge_kernel(g1, eidx, zerosD)
    g2 = _mmn(deg, acc1, g1, W2, b1r)
    acc2 = edge_kernel(g2, eidx, zerosD)
    g3 = _mmn(deg, acc2, g2, W3, b2r)
    acc3 = edge_kernel(g3, eidx, zerosD)
    return _pool(deg, acc3, g3, b3r, batchf, Wo1, bo1r, Wo2, bo2r)


# speedup vs baseline: 2.1081x; 1.0138x over previous
import functools

import jax
import jax.numpy as jnp
from jax import lax
from jax.experimental import pallas as pl
from jax.experimental.pallas import tpu as pltpu
from jax.experimental.pallas import tpu_sc as plsc

N = 10000
E = 320000
D = 128
G = 64
O = 64

NC, NS = 2, 16
NW = NC * NS
NP = 10240
RPT = NP // NS
C = 120
NCH = 84
EPT = NCH * C
EPAD = EPT * NW


@functools.cache
def _sc_kernels():
    mesh = plsc.VectorSubcoreMesh(core_axis_name="c", subcore_axis_name="s",
                                  num_cores=NC, num_subcores=NS)

    @functools.partial(
        pl.kernel,
        out_type=jax.ShapeDtypeStruct((NC, NP, D), jnp.float32),
        mesh=mesh,
        scratch_types=[
            pltpu.VMEM((2, C), jnp.int32),
            pltpu.VMEM((2, C), jnp.int32),
            pltpu.VMEM((C, D), jnp.float32),
            pltpu.SemaphoreType.DMA,
            pltpu.SemaphoreType.DMA,
            pltpu.VMEM_SHARED((NP, D), jnp.float32),
        ],
    )
    def deg_kernel(eidx_hbm, ones_hbm, zeros_hbm, out_hbm, ib0, ib1, onesv,
                   isem0, isem1, acc):
        cid = lax.axis_index("c")
        sid = lax.axis_index("s")
        pltpu.sync_copy(ones_hbm, onesv)
        pltpu.sync_copy(zeros_hbm.at[pl.ds(sid * RPT, RPT)],
                        acc.at[pl.ds(sid * RPT, RPT)])
        pltpu.async_copy(eidx_hbm.at[cid, sid, 0], ib0, isem0)
        pltpu.async_copy(eidx_hbm.at[cid, sid, 1], ib1, isem1)
        plsc.subcore_barrier()

        def half(j, ibA, isemA):
            pltpu.make_async_copy(eidx_hbm.at[cid, sid, j], ibA, isemA).wait()
            pltpu.sync_copy(onesv, acc.at[ibA.at[1]], add=True)

            @pl.when(j + 2 < NCH)
            def _():
                pltpu.async_copy(eidx_hbm.at[cid, sid, j + 2], ibA, isemA)

        def body(i, carry):
            half(2 * i, ib0, isem0)
            half(2 * i + 1, ib1, isem1)
            return carry

        lax.fori_loop(0, NCH // 2, body, 0)
        plsc.subcore_barrier()
        pltpu.sync_copy(acc.at[pl.ds(sid * RPT, RPT)],
                        out_hbm.at[cid, pl.ds(sid * RPT, RPT)])

    @functools.partial(
        pl.kernel,
        out_type=jax.ShapeDtypeStruct((NC, NP, D), jnp.float32),
        mesh=mesh,
        scratch_types=[
            [pltpu.VMEM((2, C), jnp.int32) for _ in range(3)],
            [pltpu.VMEM((C, D), jnp.float32) for _ in range(3)],
            [pltpu.SemaphoreType.DMA for _ in range(3)],
            [pltpu.SemaphoreType.DMA for _ in range(3)],
            pltpu.VMEM_SHARED((NP, D), jnp.float32),
        ],
    )
    def edge_kernel(g_hbm, eidx_hbm, zeros_hbm, out_hbm,
                    ib, rows, isem, gsem, acc):
        cid = lax.axis_index("c")
        sid = lax.axis_index("s")
        pltpu.sync_copy(zeros_hbm.at[pl.ds(sid * RPT, RPT)],
                        acc.at[pl.ds(sid * RPT, RPT)])
        pltpu.async_copy(eidx_hbm.at[cid, sid, 0], ib[0], isem[0])
        pltpu.async_copy(eidx_hbm.at[cid, sid, 1], ib[1], isem[1])
        pltpu.async_copy(eidx_hbm.at[cid, sid, 2], ib[2], isem[2])
        pltpu.make_async_copy(eidx_hbm.at[cid, sid, 0], ib[0], isem[0]).wait()
        pltpu.async_copy(g_hbm.at[ib[0].at[0]], rows[0], gsem[0])
        pltpu.make_async_copy(eidx_hbm.at[cid, sid, 1], ib[1], isem[1]).wait()
        pltpu.async_copy(g_hbm.at[ib[1].at[0]], rows[1], gsem[1])
        plsc.subcore_barrier()

        def step(j, k):
            k2 = (k + 2) % 3

            @pl.when(j + 2 < NCH)
            def _():
                pltpu.make_async_copy(eidx_hbm.at[cid, sid, j + 2], ib[k2],
                                      isem[k2]).wait()
                pltpu.async_copy(g_hbm.at[ib[k2].at[0]], rows[k2], gsem[k2])

            pltpu.make_async_copy(g_hbm.at[ib[k].at[0]], rows[k],
                                  gsem[k]).wait()
            pltpu.sync_copy(rows[k], acc.at[ib[k].at[1]], add=True)

            @pl.when(j + 3 < NCH)
            def _():
                pltpu.async_copy(eidx_hbm.at[cid, sid, j + 3], ib[k], isem[k])

        def body(i, carry):
            for k in range(3):
                step(3 * i + k, k)
            return carry

        lax.fori_loop(0, NCH // 3, body, 0)
        plsc.subcore_barrier()
        pltpu.sync_copy(acc.at[pl.ds(sid * RPT, RPT)],
                        out_hbm.at[cid, pl.ds(sid * RPT, RPT)])

    return deg_kernel, edge_kernel


BN = 1000
GRID = N // BN


def _dinv_of(deg_ref):
    d = deg_ref[0, :, 0] + deg_ref[1, :, 0] + 1.0
    return lax.rsqrt(d)[:, None]


def _mm1_body(deg_ref, x_ref, w_ref, o_ref):
    h = jnp.dot(x_ref[...], w_ref[...], preferred_element_type=jnp.float32)
    o_ref[...] = _dinv_of(deg_ref) * h


def _mmn_body(deg_ref, acc_ref, g_ref, w_ref, b_ref, o_ref):
    dinv = _dinv_of(deg_ref)
    xin = dinv * (acc_ref[0] + acc_ref[1] + g_ref[...]) + b_ref[...]
    xin = jnp.maximum(xin, 0.0)
    o_ref[...] = dinv * jnp.dot(xin, w_ref[...],
                                preferred_element_type=jnp.float32)


def _pool_body(deg_ref, acc_ref, g_ref, b_ref, batch_ref,
               wo1_ref, bo1_ref, wo2_ref, bo2_ref, o_ref,
               sums_ref, cnt_ref):
    step = pl.program_id(0)

    @pl.when(step == 0)
    def _():
        sums_ref[...] = jnp.zeros_like(sums_ref)
        cnt_ref[...] = jnp.zeros_like(cnt_ref)

    dinv = _dinv_of(deg_ref)
    h3 = dinv * (acc_ref[0] + acc_ref[1] + g_ref[...]) + b_ref[...]
    gids = lax.broadcasted_iota(jnp.int32, (1, G), 1).astype(jnp.float32)
    mask = (batch_ref[...] == gids).astype(jnp.float32)
    sums_ref[...] += lax.dot_general(
        mask, h3, (((0,), (0,)), ((), ())),
        preferred_element_type=jnp.float32)
    cnt_ref[...] += jnp.sum(mask, axis=0)[None, :]

    @pl.when(step == GRID - 1)
    def _():
        pooled = sums_ref[...] / jnp.maximum(cnt_ref[...], 1.0)[0][:, None]
        z = jnp.maximum(
            jnp.dot(pooled, wo1_ref[...],
                    preferred_element_type=jnp.float32) + bo1_ref[...], 0.0)
        o_ref[...] = jnp.dot(z, wo2_ref[...],
                             preferred_element_type=jnp.float32) + bo2_ref[...]


_deg_spec = pl.BlockSpec((NC, BN, D), lambda i: (0, i, 0))
_acc_spec = pl.BlockSpec((NC, BN, D), lambda i: (0, i, 0))
_row_spec = pl.BlockSpec((BN, D), lambda i: (i, 0))


def _full(shape):
    return pl.BlockSpec(shape, lambda i: tuple(0 for _ in shape))


_mm1 = pl.pallas_call(
    _mm1_body,
    grid=(GRID,),
    in_specs=[_deg_spec, _row_spec, _full((D, D))],
    out_specs=_row_spec,
    out_shape=jax.ShapeDtypeStruct((N, D), jnp.float32),
)

_mmn = pl.pallas_call(
    _mmn_body,
    grid=(GRID,),
    in_specs=[_deg_spec, _acc_spec, _row_spec, _full((D, D)), _full((1, D))],
    out_specs=_row_spec,
    out_shape=jax.ShapeDtypeStruct((N, D), jnp.float32),
)

_pool = pl.pallas_call(
    _pool_body,
    grid=(GRID,),
    in_specs=[_deg_spec, _acc_spec, _row_spec, _full((1, D)),
              pl.BlockSpec((BN, 1), lambda i: (i, 0)),
              _full((D, G)), _full((1, G)), _full((G, O)), _full((1, O))],
    out_specs=_full((G, O)),
    out_shape=jax.ShapeDtypeStruct((G, O), jnp.float32),
    scratch_shapes=[pltpu.VMEM((G, D), jnp.float32),
                    pltpu.VMEM((1, G), jnp.float32)],
)


def kernel(x, edge_index, batch, W1, b1, W2, b2, W3, b3, Wo1, bo1, Wo2, bo2):
    src = edge_index[0].astype(jnp.int32)
    dst = edge_index[1].astype(jnp.int32)
    pad = EPAD - E
    src = jnp.concatenate([src, jnp.zeros((pad,), jnp.int32)])
    dst = jnp.concatenate([dst, jnp.full((pad,), NP - 1, jnp.int32)])
    eidx = jnp.stack([src.reshape(NC, NS, NCH, C),
                      dst.reshape(NC, NS, NCH, C)], axis=3)

    zerosD = jnp.zeros((NP, D), jnp.float32)
    batchf = batch.astype(jnp.float32).reshape(N, 1)
    b1r = b1.reshape(1, D)
    b2r = b2.reshape(1, D)
    b3r = b3.reshape(1, D)
    bo1r = bo1.reshape(1, G)
    bo2r = bo2.reshape(1, O)

    deg_kernel, edge_kernel = _sc_kernels()
    deg = deg_kernel(eidx, jnp.ones((C, D), jnp.float32), zerosD)
    g1 = _mm1(deg, x, W1)
    acc1 = edge_kernel(g1, eidx, zerosD)
    g2 = _mmn(deg, acc1, g1, W2, b1r)
    acc2 = edge_kernel(g2, eidx, zerosD)
    g3 = _mmn(deg, acc2, g2, W3, b2r)
    acc3 = edge_kernel(g3, eidx, zerosD)
    return _pool(deg, acc3, g3, b3r, batchf, Wo1, bo1r, Wo2, bo2r)
